# Initial kernel scaffold; baseline (speedup 1.0000x reference)
import jax
import jax.numpy as jnp
from jax.experimental import pallas as pl


def kernel(type, location, source_location, time, start_time, end_time, request_time, commit_time, demand, material, method, quantity, parent, child, type_table, loc_table, time_table, demand_table, mat_table, method_table, Wq, bq, gamma, beta):
    raise NotImplementedError("write your pallas kernel here")


# speedup vs baseline: 25.2854x; 25.2854x over previous
import jax
import jax.numpy as jnp
from jax import lax
from jax.experimental import pallas as pl
from jax.experimental.pallas import tpu as pltpu

_D = 128
_T = 512


def _body(ty_ref, lo_ref, sl_ref, tm_ref, st_ref, en_ref, rq_ref, cm_ref,
          dm_ref, mt_ref, me_ref, pa_ref, ch_ref, q_ref,
          big_ref, wt_ref, bt_ref, gt_ref, bet_ref, o_ref):
  f32 = jnp.float32
  bf16 = jnp.bfloat16
  i16 = jnp.int16
  c16 = lax.broadcasted_iota(jnp.int32, (_D, _T), 0).astype(i16)
  zero_b = jnp.zeros((_D, _T), bf16)

  def rows(ref):
    r8 = jnp.broadcast_to(ref[0], (8, _T)).astype(i16)
    return jnp.concatenate([r8] * 16, axis=0)

  ty128 = rows(ty_ref)
  nb_b = jnp.where(ty128 == 7, zero_b, jnp.full((_D, _T), 1, bf16))
  isb_b = jnp.where(ty128 == 7, jnp.full((_D, _T), 1, bf16), zero_b)

  def oh(idx128, off, sel):
    return jnp.where(c16 == idx128 + i16(off), sel, zero_b)

  mh_a = (oh(ty128, 0, nb_b) + oh(rows(lo_ref), 8, nb_b)
          + oh(rows(sl_ref), 8, nb_b) + oh(rows(dm_ref), 18, nb_b))
  mh_t = (oh(rows(tm_ref), 0, nb_b) + oh(rows(st_ref), 0, nb_b)
          + oh(rows(en_ref), 0, nb_b) + oh(rows(rq_ref), 0, nb_b)
          + oh(rows(cm_ref), 0, nb_b))
  mh_m = oh(rows(mt_ref), 0, nb_b)
  me128 = rows(me_ref)
  mh_b = oh(rows(pa_ref), 0, isb_b) + oh(rows(ch_ref), 0, isb_b)
  big_mh = jnp.concatenate(
      [mh_a, mh_t, mh_m] + [oh(me128, -k * _D, nb_b) for k in range(5)]
      + [mh_b], axis=0)

  acc = lax.dot_general(
      big_mh, big_ref[...], (((0,), (0,)), ((), ())),
      preferred_element_type=f32)

  q8 = jnp.broadcast_to(q_ref[0], (8, _T))
  qt = jnp.concatenate([q8] * 16, axis=0)
  h = jnp.maximum(qt * wt_ref[...] + bt_ref[...], 0.0)
  mu = jnp.mean(h, axis=0, keepdims=True)
  mu_t = jnp.concatenate([jnp.broadcast_to(mu, (8, _T))] * 16, axis=0)
  hc = h - mu_t
  var = jnp.mean(hc * hc, axis=0, keepdims=True)
  rs = lax.rsqrt(var + 1e-5)
  rs_t = jnp.concatenate([jnp.broadcast_to(rs, (8, _T))] * 16, axis=0)
  e_q = (hc * rs_t * gt_ref[...] + bet_ref[...]) * nb_b.astype(f32)

  o_ref[...] = acc + jnp.transpose(e_q, (1, 0))


@jax.jit
def kernel(type, location, source_location, time, start_time, end_time,
           request_time, commit_time, demand, material, method, quantity,
           parent, child, type_table, loc_table, time_table, demand_table,
           mat_table, method_table, Wq, bq, gamma, beta):
  B, L = type.shape
  N = B * L
  nb = N // _T
  assert N % _T == 0
  bf16 = jnp.bfloat16

  def prep(x):
    return x.reshape(nb, 1, _T)

  def padrows(tab, rows):
    return jnp.pad(tab, ((0, rows - tab.shape[0]), (0, 0)))

  ga_tab = jnp.concatenate(
      [type_table, loc_table, demand_table,
       jnp.zeros((_D - 68, _D), jnp.float32)], axis=0)
  big_tab = jnp.concatenate(
      [ga_tab, padrows(time_table, _D), padrows(mat_table, _D),
       padrows(method_table, 640), padrows(mat_table, _D)],
      axis=0).astype(bf16)

  def tcol(v):
    return jnp.broadcast_to(v.reshape(_D, 1), (_D, _T))

  row_spec = pl.BlockSpec((1, 1, _T), lambda i: (i, 0, 0))
  const_t_spec = pl.BlockSpec((_D, _T), lambda i: (0, 0))

  args = (
      prep(type), prep(location), prep(source_location), prep(time),
      prep(start_time), prep(end_time), prep(request_time),
      prep(commit_time), prep(demand), prep(material), prep(method),
      prep(parent), prep(child), prep(quantity),
      big_tab, tcol(Wq.reshape(_D)), tcol(bq), tcol(gamma), tcol(beta),
  )

  out = pl.pallas_call(
      _body,
      grid=(nb,),
      in_specs=[row_spec] * 14
      + [pl.BlockSpec((1152, _D), lambda i: (0, 0))]
      + [const_t_spec] * 4,
      out_specs=pl.BlockSpec((_T, _D), lambda i: (i, 0)),
      out_shape=jax.ShapeDtypeStruct((N, _D), jnp.float32),
      compiler_params=pltpu.CompilerParams(
          fuse_transposed_lhs_in_matmul=True),
  )(*args)
  return out.reshape(B, L, _D)


# speedup vs baseline: 30.4444x; 1.2040x over previous
import jax
import jax.numpy as jnp
from jax import lax
from jax.experimental import pallas as pl
from jax.experimental.pallas import tpu as pltpu

_D = 128
_T = 1024


def _body(ty_ref, lo_ref, sl_ref, tm_ref, st_ref, en_ref, rq_ref, cm_ref,
          dm_ref, mt_ref, me_ref, pa_ref, ch_ref, q_ref,
          big_ref, wt_ref, bt_ref, gt_ref, bet_ref, o_ref):
  f32 = jnp.float32
  bf16 = jnp.bfloat16
  i16 = jnp.int16
  c16 = lax.broadcasted_iota(jnp.int32, (_D, _T), 0).astype(i16)
  zero_b = jnp.zeros((_D, _T), bf16)

  def rows(ref):
    r8 = jnp.broadcast_to(ref[0], (8, _T)).astype(i16)
    return jnp.concatenate([r8] * 16, axis=0)

  ty128 = rows(ty_ref)
  nb_b = jnp.where(ty128 == 7, zero_b, jnp.full((_D, _T), 1, bf16))
  isb_b = jnp.where(ty128 == 7, jnp.full((_D, _T), 1, bf16), zero_b)

  def oh(idx128, off, sel):
    return jnp.where(c16 == idx128 + i16(off), sel, zero_b)

  mh_a = (oh(ty128, 0, nb_b) + oh(rows(lo_ref), 8, nb_b)
          + oh(rows(sl_ref), 8, nb_b) + oh(rows(dm_ref), 18, nb_b))
  mh_t = (oh(rows(tm_ref), 0, nb_b) + oh(rows(st_ref), 0, nb_b)
          + oh(rows(en_ref), 0, nb_b) + oh(rows(rq_ref), 0, nb_b)
          + oh(rows(cm_ref), 0, nb_b))
  mh_m = oh(rows(mt_ref), 0, nb_b)
  me128 = rows(me_ref)
  mh_b = oh(rows(pa_ref), 0, isb_b) + oh(rows(ch_ref), 0, isb_b)
  big_mh = jnp.concatenate(
      [mh_a, mh_t, mh_m] + [oh(me128, -k * _D, nb_b) for k in range(5)]
      + [mh_b], axis=0)

  acc = lax.dot_general(
      big_mh, big_ref[...], (((0,), (0,)), ((), ())),
      preferred_element_type=f32)

  q8 = jnp.broadcast_to(q_ref[0], (8, _T))
  qt = jnp.concatenate([q8] * 16, axis=0)
  h = jnp.maximum(qt * wt_ref[...] + bt_ref[...], 0.0)
  mu = jnp.mean(h, axis=0, keepdims=True)
  mu_t = jnp.concatenate([jnp.broadcast_to(mu, (8, _T))] * 16, axis=0)
  hc = h - mu_t
  var = jnp.mean(hc * hc, axis=0, keepdims=True)
  rs = lax.rsqrt(var + 1e-5)
  rs_t = jnp.concatenate([jnp.broadcast_to(rs, (8, _T))] * 16, axis=0)
  e_q = (hc * rs_t * gt_ref[...] + bet_ref[...]) * nb_b.astype(f32)

  o_ref[...] = acc + jnp.transpose(e_q, (1, 0))


@jax.jit
def kernel(type, location, source_location, time, start_time, end_time,
           request_time, commit_time, demand, material, method, quantity,
           parent, child, type_table, loc_table, time_table, demand_table,
           mat_table, method_table, Wq, bq, gamma, beta):
  B, L = type.shape
  N = B * L
  nb = N // _T
  assert N % _T == 0
  bf16 = jnp.bfloat16

  def prep(x):
    return x.reshape(nb, 1, _T)

  def padrows(tab, rows):
    return jnp.pad(tab, ((0, rows - tab.shape[0]), (0, 0)))

  ga_tab = jnp.concatenate(
      [type_table, loc_table, demand_table,
       jnp.zeros((_D - 68, _D), jnp.float32)], axis=0)
  big_tab = jnp.concatenate(
      [ga_tab, padrows(time_table, _D), padrows(mat_table, _D),
       padrows(method_table, 640), padrows(mat_table, _D)],
      axis=0).astype(bf16)

  def tcol(v):
    return jnp.broadcast_to(v.reshape(_D, 1), (_D, _T))

  row_spec = pl.BlockSpec((1, 1, _T), lambda i: (i, 0, 0))
  const_t_spec = pl.BlockSpec((_D, _T), lambda i: (0, 0))

  args = (
      prep(type), prep(location), prep(source_location), prep(time),
      prep(start_time), prep(end_time), prep(request_time),
      prep(commit_time), prep(demand), prep(material), prep(method),
      prep(parent), prep(child), prep(quantity),
      big_tab, tcol(Wq.reshape(_D)), tcol(bq), tcol(gamma), tcol(beta),
  )

  out = pl.pallas_call(
      _body,
      grid=(nb,),
      in_specs=[row_spec] * 14
      + [pl.BlockSpec((1152, _D), lambda i: (0, 0))]
      + [const_t_spec] * 4,
      out_specs=pl.BlockSpec((_T, _D), lambda i: (i, 0)),
      out_shape=jax.ShapeDtypeStruct((N, _D), jnp.float32),
      compiler_params=pltpu.CompilerParams(
          fuse_transposed_lhs_in_matmul=True),
  )(*args)
  return out.reshape(B, L, _D)


# speedup vs baseline: 32.4549x; 1.0660x over previous
import jax
import jax.numpy as jnp
from jax import lax
from jax.experimental import pallas as pl
from jax.experimental.pallas import tpu as pltpu

_D = 128
_T = 2048


def _body(ty_ref, lo_ref, sl_ref, tm_ref, st_ref, en_ref, rq_ref, cm_ref,
          dm_ref, mt_ref, me_ref, pa_ref, ch_ref, q_ref,
          big_ref, wt_ref, bt_ref, gt_ref, bet_ref, o_ref):
  f32 = jnp.float32
  bf16 = jnp.bfloat16
  i16 = jnp.int16
  c16 = lax.broadcasted_iota(jnp.int32, (_D, _T), 0).astype(i16)
  zero_b = jnp.zeros((_D, _T), bf16)

  def rows(ref):
    r8 = jnp.broadcast_to(ref[0], (8, _T)).astype(i16)
    return jnp.concatenate([r8] * 16, axis=0)

  ty128 = rows(ty_ref)
  nb_b = jnp.where(ty128 == 7, zero_b, jnp.full((_D, _T), 1, bf16))
  isb_b = jnp.where(ty128 == 7, jnp.full((_D, _T), 1, bf16), zero_b)

  def oh(idx128, off, sel):
    return jnp.where(c16 == idx128 + i16(off), sel, zero_b)

  mh_a = (oh(ty128, 0, nb_b) + oh(rows(lo_ref), 8, nb_b)
          + oh(rows(sl_ref), 8, nb_b) + oh(rows(dm_ref), 18, nb_b))
  mh_t = (oh(rows(tm_ref), 0, nb_b) + oh(rows(st_ref), 0, nb_b)
          + oh(rows(en_ref), 0, nb_b) + oh(rows(rq_ref), 0, nb_b)
          + oh(rows(cm_ref), 0, nb_b))
  mh_m = oh(rows(mt_ref), 0, nb_b)
  me128 = rows(me_ref)
  mh_b = oh(rows(pa_ref), 0, isb_b) + oh(rows(ch_ref), 0, isb_b)
  big_mh = jnp.concatenate(
      [mh_a, mh_t, mh_m] + [oh(me128, -k * _D, nb_b) for k in range(5)]
      + [mh_b], axis=0)

  acc = lax.dot_general(
      big_mh, big_ref[...], (((0,), (0,)), ((), ())),
      preferred_element_type=f32)

  q8 = jnp.broadcast_to(q_ref[0], (8, _T))
  qt = jnp.concatenate([q8] * 16, axis=0)
  h = jnp.maximum(qt * wt_ref[...] + bt_ref[...], 0.0)
  mu = jnp.mean(h, axis=0, keepdims=True)
  mu_t = jnp.concatenate([jnp.broadcast_to(mu, (8, _T))] * 16, axis=0)
  hc = h - mu_t
  var = jnp.mean(hc * hc, axis=0, keepdims=True)
  rs = lax.rsqrt(var + 1e-5)
  rs_t = jnp.concatenate([jnp.broadcast_to(rs, (8, _T))] * 16, axis=0)
  e_q = (hc * rs_t * gt_ref[...] + bet_ref[...]) * nb_b.astype(f32)

  o_ref[...] = acc + jnp.transpose(e_q, (1, 0))


@jax.jit
def kernel(type, location, source_location, time, start_time, end_time,
           request_time, commit_time, demand, material, method, quantity,
           parent, child, type_table, loc_table, time_table, demand_table,
           mat_table, method_table, Wq, bq, gamma, beta):
  B, L = type.shape
  N = B * L
  nb = N // _T
  assert N % _T == 0
  bf16 = jnp.bfloat16

  def prep(x):
    return x.reshape(nb, 1, _T)

  def padrows(tab, rows):
    return jnp.pad(tab, ((0, rows - tab.shape[0]), (0, 0)))

  ga_tab = jnp.concatenate(
      [type_table, loc_table, demand_table,
       jnp.zeros((_D - 68, _D), jnp.float32)], axis=0)
  big_tab = jnp.concatenate(
      [ga_tab, padrows(time_table, _D), padrows(mat_table, _D),
       padrows(method_table, 640), padrows(mat_table, _D)],
      axis=0).astype(bf16)

  def tcol(v):
    return jnp.broadcast_to(v.reshape(_D, 1), (_D, _T))

  row_spec = pl.BlockSpec((1, 1, _T), lambda i: (i, 0, 0))
  const_t_spec = pl.BlockSpec((_D, _T), lambda i: (0, 0))

  args = (
      prep(type), prep(location), prep(source_location), prep(time),
      prep(start_time), prep(end_time), prep(request_time),
      prep(commit_time), prep(demand), prep(material), prep(method),
      prep(parent), prep(child), prep(quantity),
      big_tab, tcol(Wq.reshape(_D)), tcol(bq), tcol(gamma), tcol(beta),
  )

  out = pl.pallas_call(
      _body,
      grid=(nb,),
      in_specs=[row_spec] * 14
      + [pl.BlockSpec((1152, _D), lambda i: (0, 0))]
      + [const_t_spec] * 4,
      out_specs=pl.BlockSpec((_T, _D), lambda i: (i, 0)),
      out_shape=jax.ShapeDtypeStruct((N, _D), jnp.float32),
      compiler_params=pltpu.CompilerParams(
          fuse_transposed_lhs_in_matmul=True),
  )(*args)
  return out.reshape(B, L, _D)
